# baseline (speedup 1.0000x reference)
import functools

import jax
import jax.numpy as jnp
from jax import lax
from jax.experimental import pallas as pl
from jax.experimental.pallas import tpu as pltpu
from jax.experimental.pallas import tpu_sc as plsc

NC = 2
NS = 16
L = 16
NW = NC * NS
B = 16384
BPW = B // NW
NCHUNK = 4
CW = BPW // NCHUNK
NGRP = CW // L
D = 16
NSTUD = 1_000_000
COLW = 65_536

WBW = 512
WB_PER_W = 12
NSC = NW * WB_PER_W * WBW
NTC = NSTUD - NSC


def _sc_colsum_body(thetaT_hbm, out_hbm, buf0, buf1, ob0, ob1,
                    sr0, sr1, sw0, sw1):
    wid = lax.axis_index("s") * NC + lax.axis_index("c")
    first = wid * WB_PER_W

    def fire_read(blk, buf, sem):
        col = pl.multiple_of(blk * WBW, 128)
        pltpu.async_copy(thetaT_hbm.at[:, pl.ds(col, WBW)], buf, sem)

    def wait_read(buf, sem):
        pltpu.make_async_copy(thetaT_hbm.at[:, pl.ds(0, WBW)], buf, sem).wait()

    def reduce_block(buf, ob):
        for grp in range(WBW // L):
            v = buf[0, pl.ds(grp * L, L)]
            for f in range(1, D):
                v = v + buf[f, pl.ds(grp * L, L)]
            ob[pl.ds(grp * L, L)] = v

    def fire_write(blk, ob, sem):
        col = pl.multiple_of(blk * WBW, 128)
        pltpu.async_copy(ob, out_hbm.at[pl.ds(col, WBW)], sem)

    def wait_write(ob, sem):
        pltpu.make_async_copy(thetaT_hbm.at[0, pl.ds(0, WBW)], ob, sem).wait()

    fire_read(first, buf0, sr0)

    def body(g, carry):
        blk = first + 2 * g
        fire_read(blk + 1, buf1, sr1)
        wait_read(buf0, sr0)
        reduce_block(buf0, ob0)

        @pl.when(g > 0)
        def _():
            wait_write(ob0, sw0)
        fire_write(blk, ob0, sw0)
        fire_read(blk + 2, buf0, sr0)

        wait_read(buf1, sr1)
        reduce_block(buf1, ob1)

        @pl.when(g > 0)
        def _():
            wait_write(ob1, sw1)
        fire_write(blk + 1, ob1, sw1)
        return carry

    lax.fori_loop(0, WB_PER_W // 2, body, 0)
    wait_read(buf0, sr0)
    wait_write(ob0, sw0)
    wait_write(ob1, sw1)


@functools.cache
def _build_sc_colsum():
    return pl.kernel(
        _sc_colsum_body,
        mesh=plsc.VectorSubcoreMesh(core_axis_name="c", subcore_axis_name="s"),
        compiler_params=pltpu.CompilerParams(
            needs_layout_passes=False, use_tc_tiling_on_sc=True),
        cost_estimate=pl.CostEstimate(
            flops=16 * NSC, transcendentals=0,
            bytes_accessed=4 * D * NSC + 4 * NSC),
        out_type=jax.ShapeDtypeStruct((NSC,), jnp.float32),
        scratch_types=[
            pltpu.VMEM((D, WBW), jnp.float32),
            pltpu.VMEM((D, WBW), jnp.float32),
            pltpu.VMEM((WBW,), jnp.float32),
            pltpu.VMEM((WBW,), jnp.float32),
            pltpu.SemaphoreType.DMA,
            pltpu.SemaphoreType.DMA,
            pltpu.SemaphoreType.DMA,
            pltpu.SemaphoreType.DMA,
        ],
    )


def _colsum_body(x_ref, o_ref):
    o_ref[...] = jnp.sum(x_ref[...], axis=0)


def _colsum(thetaT):
    grid = (NTC + COLW - 1) // COLW
    skip = NSC // COLW
    return pl.pallas_call(
        _colsum_body,
        grid=(grid,),
        in_specs=[pl.BlockSpec((D, COLW), lambda i: (0, i + skip))],
        out_specs=pl.BlockSpec((COLW,), lambda i: (i,)),
        out_shape=jax.ShapeDtypeStruct((NTC,), jnp.float32),
    )(thetaT)


def _irt_body(sid_hbm, qid_hbm, csum_hbm, beta_hbm, out_hbm,
              sidx, qidx, cvals, bvals, outv, sem_c, sem_b):
    wid = lax.axis_index("s") * NC + lax.axis_index("c")
    pltpu.sync_copy(sid_hbm.at[wid], sidx)
    pltpu.sync_copy(qid_hbm.at[wid], qidx)

    ccopies = [pltpu.async_copy(csum_hbm.at[sidx.at[c]], cvals.at[c], sem_c)
               for c in range(NCHUNK)]
    bcopies = [pltpu.async_copy(beta_hbm.at[qidx.at[c]], bvals.at[c], sem_b)
               for c in range(NCHUNK)]
    for cp in ccopies:
        cp.wait()
    for cp in bcopies:
        cp.wait()

    for c in range(NCHUNK):
        for g in range(NGRP):
            sl = pl.ds(g * L, L)
            x = cvals[c, sl] - bvals[c, sl]
            outv[c, sl] = 1.0 / (1.0 + jnp.exp(-x))

    pltpu.sync_copy(outv, out_hbm.at[wid])


@functools.cache
def _build_irt_call():
    return pl.kernel(
        _irt_body,
        mesh=plsc.VectorSubcoreMesh(core_axis_name="c", subcore_axis_name="s"),
        compiler_params=pltpu.CompilerParams(
            needs_layout_passes=False, use_tc_tiling_on_sc=False),
        out_type=jax.ShapeDtypeStruct((NW, NCHUNK, CW), jnp.float32),
        scratch_types=[
            pltpu.VMEM((NCHUNK, CW), jnp.int32),
            pltpu.VMEM((NCHUNK, CW), jnp.int32),
            pltpu.VMEM((NCHUNK, CW), jnp.float32),
            pltpu.VMEM((NCHUNK, CW), jnp.float32),
            pltpu.VMEM((NCHUNK, CW), jnp.float32),
            pltpu.SemaphoreType.DMA,
            pltpu.SemaphoreType.DMA,
        ],
    )


def kernel(student_ids, question_ids, theta_table, beta_table):
    sid = student_ids.astype(jnp.int32).reshape(NW, NCHUNK, CW)
    qid = question_ids.astype(jnp.int32).reshape(NW, NCHUNK, CW)
    thetaT = theta_table.T
    csum_sc = _build_sc_colsum()(thetaT)
    csum_tc = _colsum(thetaT)
    colsum = jnp.concatenate([csum_sc, csum_tc])
    beta_flat = beta_table[:, 0]
    out = _build_irt_call()(sid, qid, colsum, beta_flat)
    return out.reshape(B, 1)


# speedup vs baseline: 1.1891x; 1.1891x over previous
import functools

import jax
import jax.numpy as jnp
from jax import lax
from jax.experimental import pallas as pl
from jax.experimental.pallas import tpu as pltpu
from jax.experimental.pallas import tpu_sc as plsc

NC = 2
NS = 16
L = 16
NW = NC * NS
B = 16384
BPW = B // NW
NCHUNK = 4
CW = BPW // NCHUNK
NGRP = CW // L
D = 16
NSTUD = 1_000_000
COLW = 131_072


def _colsum_body(x_ref, o_ref):
    o_ref[...] = jnp.sum(x_ref[...], axis=0)


def _colsum(thetaT):
    grid = (NSTUD + COLW - 1) // COLW
    return pl.pallas_call(
        _colsum_body,
        grid=(grid,),
        in_specs=[pl.BlockSpec((D, COLW), lambda i: (0, i))],
        out_specs=pl.BlockSpec((COLW,), lambda i: (i,)),
        out_shape=jax.ShapeDtypeStruct((NSTUD,), jnp.float32),
    )(thetaT)


def _irt_body(sid_hbm, qid_hbm, csum_hbm, beta_hbm, out_hbm,
              sidx, qidx, cvals, bvals, outv, sem_c, sem_b):
    wid = lax.axis_index("s") * NC + lax.axis_index("c")
    pltpu.sync_copy(sid_hbm.at[wid], sidx)
    pltpu.sync_copy(qid_hbm.at[wid], qidx)

    ccopies = [pltpu.async_copy(csum_hbm.at[sidx.at[c]], cvals.at[c], sem_c)
               for c in range(NCHUNK)]
    bcopies = [pltpu.async_copy(beta_hbm.at[qidx.at[c]], bvals.at[c], sem_b)
               for c in range(NCHUNK)]
    for cp in ccopies:
        cp.wait()
    for cp in bcopies:
        cp.wait()

    for c in range(NCHUNK):
        for g in range(NGRP):
            sl = pl.ds(g * L, L)
            x = cvals[c, sl] - bvals[c, sl]
            outv[c, sl] = 1.0 / (1.0 + jnp.exp(-x))

    pltpu.sync_copy(outv, out_hbm.at[wid])


@functools.cache
def _build_irt_call():
    return pl.kernel(
        _irt_body,
        mesh=plsc.VectorSubcoreMesh(core_axis_name="c", subcore_axis_name="s"),
        compiler_params=pltpu.CompilerParams(
            needs_layout_passes=False, use_tc_tiling_on_sc=False),
        out_type=jax.ShapeDtypeStruct((NW, NCHUNK, CW), jnp.float32),
        scratch_types=[
            pltpu.VMEM((NCHUNK, CW), jnp.int32),
            pltpu.VMEM((NCHUNK, CW), jnp.int32),
            pltpu.VMEM((NCHUNK, CW), jnp.float32),
            pltpu.VMEM((NCHUNK, CW), jnp.float32),
            pltpu.VMEM((NCHUNK, CW), jnp.float32),
            pltpu.SemaphoreType.DMA,
            pltpu.SemaphoreType.DMA,
        ],
    )


def kernel(student_ids, question_ids, theta_table, beta_table):
    sid = student_ids.astype(jnp.int32).reshape(NW, NCHUNK, CW)
    qid = question_ids.astype(jnp.int32).reshape(NW, NCHUNK, CW)
    colsum = _colsum(theta_table.T)
    beta_flat = beta_table[:, 0]
    out = _build_irt_call()(sid, qid, colsum, beta_flat)
    return out.reshape(B, 1)
